# Initial kernel scaffold; baseline (speedup 1.0000x reference)
import jax
import jax.numpy as jnp
from jax.experimental import pallas as pl


def kernel(x, edge_index_0, edge_index_1, W_l0, b0, W_r0, W_l1, b1, W_r1, att, W_out, b_out):
    raise NotImplementedError("write your pallas kernel here")


# speedup vs baseline: 3.5395x; 3.5395x over previous
import functools

import jax
import jax.numpy as jnp
from jax import lax
from jax.experimental import pallas as pl
from jax.experimental.pallas import tpu as pltpu
from jax.experimental.pallas import tpu_sc as plsc

F32 = jnp.float32

_N0, _N1, _N2 = 16384, 4096, 1024
_D = 512
_G = 128
_B = 128
_NSUB = 16


def _make_sc_agg(n_edges, n_tgt):
  e_per_tile = n_edges // _NSUB
  n_batches = e_per_tile // _B
  rows_per_tile = n_tgt // _NSUB

  mesh = plsc.VectorSubcoreMesh(core_axis_name="c", subcore_axis_name="s")

  @functools.partial(
      pl.kernel,
      mesh=mesh,
      out_type=(
          jax.ShapeDtypeStruct((4, n_tgt, _G), F32),
          jax.ShapeDtypeStruct((2, _NSUB, n_tgt // 16, 16), F32),
      ),
      scratch_types=[
          pltpu.VMEM_SHARED((n_tgt, _G), F32),
          pltpu.VMEM((_B, _G), F32),
          pltpu.VMEM((n_tgt // 16, 16), F32),
          pltpu.VMEM((_B,), jnp.int32),
          pltpu.VMEM((_B,), jnp.int32),
          pltpu.VMEM((_B,), jnp.int32),
          pltpu.SemaphoreType.DMA,
      ],
  )
  def agg(x4, src, dst, out_sum, out_cnt, acc, rows_v, cntp,
          src_v, dst_v, idx_v, sem):
    c = lax.axis_index("c")
    s = lax.axis_index("s")

    zero16 = jnp.zeros((16,), F32)
    lane16 = lax.iota(jnp.int32, 16)

    def _zrow(r, carry):
      for j in range(_G // 16):
        rows_v[r, pl.ds(j * 16, 16)] = zero16
      return carry

    lax.fori_loop(0, _B, _zrow, 0)

    def _zcnt(r, carry):
      cntp[r, :] = zero16
      return carry

    lax.fori_loop(0, n_tgt // 16, _zcnt, 0)

    r0 = s * rows_per_tile

    for gpass in range(2):
      g = 2 * c + gpass

      for k in range(0, rows_per_tile, _B):
        blk = min(_B, rows_per_tile - k)
        pltpu.sync_copy(rows_v.at[pl.ds(0, blk)], acc.at[pl.ds(r0 + k, blk)])

      plsc.subcore_barrier()

      def _batch(t, carry):
        base = s * e_per_tile + t * _B
        pltpu.sync_copy(src.at[pl.ds(base, _B)], src_v)
        pltpu.sync_copy(dst.at[pl.ds(base, _B)], dst_v)
        for j in range(_B // 16):
          sl = pl.ds(j * 16, 16)
          idx_v[sl] = src_v[sl] * 4 + g
        pltpu.async_copy(x4.at[idx_v], rows_v, sem).wait()
        pltpu.sync_copy(rows_v, acc.at[dst_v], add=True)

        if gpass == 0:
          @pl.when(t % 2 == c)
          def _count():
            def _chunk(j, carry2):
              dv = dst_v[pl.ds(j * 16, 16)]
              for l in range(16):
                d = dv[l]
                row = d >> 4
                oh = jnp.where(lane16 == (d & 15), 1.0, 0.0).astype(F32)
                cntp[row, :] = cntp[row, :] + oh
              return carry2

            lax.fori_loop(0, _B // 16, _chunk, 0)

      def _batch_wrap(t, carry):
        _batch(t, carry)
        return carry

      lax.fori_loop(0, n_batches, _batch_wrap, 0)

      plsc.subcore_barrier()

      pltpu.sync_copy(acc.at[pl.ds(r0, rows_per_tile)],
                      out_sum.at[g, pl.ds(r0, rows_per_tile), :])

      if gpass == 0:
        def _rezero(r, carry):
          for j in range(_G // 16):
            rows_v[r, pl.ds(j * 16, 16)] = zero16
          return carry

        lax.fori_loop(0, _B, _rezero, 0)

    pltpu.sync_copy(cntp, out_cnt.at[c, s])

  return agg


_agg0 = _make_sc_agg(_N1 * 16, _N1)
_agg1 = _make_sc_agg(_N2 * 16, _N2)


def _sage_dense(summed4, cnt32, x_tgt, w_l, w_r, b, bm):
  n_tgt, d_in = x_tgt.shape
  d_out = w_l.shape[1]

  def body(s_ref, c_ref, xt_ref, wl_ref, wr_ref, b_ref, o_ref):
    cnt = jnp.maximum(jnp.sum(c_ref[...], axis=0), 1.0)[:, None]
    acc = jnp.dot(xt_ref[...], wr_ref[...], preferred_element_type=F32)
    for g in range(4):
      acc += jnp.dot(s_ref[g] / cnt, wl_ref[pl.ds(g * _G, _G), :],
                     preferred_element_type=F32)
    o_ref[...] = jnp.maximum(acc + b_ref[...], 0.0)

  return pl.pallas_call(
      body,
      grid=(n_tgt // bm,),
      in_specs=[
          pl.BlockSpec((4, bm, _G), lambda i: (0, i, 0)),
          pl.BlockSpec((32, bm), lambda i: (0, i)),
          pl.BlockSpec((bm, d_in), lambda i: (i, 0)),
          pl.BlockSpec((d_in, d_out), lambda i: (0, 0)),
          pl.BlockSpec((d_in, d_out), lambda i: (0, 0)),
          pl.BlockSpec((1, d_out), lambda i: (0, 0)),
      ],
      out_specs=pl.BlockSpec((bm, d_out), lambda i: (i, 0)),
      out_shape=jax.ShapeDtypeStruct((n_tgt, d_out), F32),
  )(summed4, cnt32, x_tgt, w_l, w_r, b)


def _final(summed4, cnt32, h0c, w_l, w_r, b, att, w_out, b_out, bm):
  n_tgt, d_in = h0c.shape
  d_out = w_out.shape[1]

  def body(s_ref, c_ref, h0_ref, wl_ref, wr_ref, b_ref, att_ref, wo_ref,
           bo_ref, o_ref):
    cnt = jnp.maximum(jnp.sum(c_ref[...], axis=0), 1.0)[:, None]
    h0b = h0_ref[...]
    acc = jnp.dot(h0b, wr_ref[...], preferred_element_type=F32)
    for g in range(4):
      acc += jnp.dot(s_ref[g] / cnt, wl_ref[pl.ds(g * _G, _G), :],
                     preferred_element_type=F32)
    h1 = jnp.maximum(acc + b_ref[...], 0.0)
    a = att_ref[...]
    sc0 = jnp.sum(jnp.tanh(h0b) * a, axis=1, keepdims=True)
    sc1 = jnp.sum(jnp.tanh(h1) * a, axis=1, keepdims=True)
    m = jnp.maximum(sc0, sc1)
    e0 = jnp.exp(sc0 - m)
    e1 = jnp.exp(sc1 - m)
    inv = 1.0 / (e0 + e1)
    h = (e0 * inv) * h0b + (e1 * inv) * h1
    o_ref[...] = jnp.dot(h, wo_ref[...], preferred_element_type=F32) + bo_ref[...]

  return pl.pallas_call(
      body,
      grid=(n_tgt // bm,),
      in_specs=[
          pl.BlockSpec((4, bm, _G), lambda i: (0, i, 0)),
          pl.BlockSpec((32, bm), lambda i: (0, i)),
          pl.BlockSpec((bm, d_in), lambda i: (i, 0)),
          pl.BlockSpec((d_in, d_in), lambda i: (0, 0)),
          pl.BlockSpec((d_in, d_in), lambda i: (0, 0)),
          pl.BlockSpec((1, d_in), lambda i: (0, 0)),
          pl.BlockSpec((1, d_in), lambda i: (0, 0)),
          pl.BlockSpec((d_in, d_out), lambda i: (0, 0)),
          pl.BlockSpec((1, d_out), lambda i: (0, 0)),
      ],
      out_specs=pl.BlockSpec((bm, d_out), lambda i: (i, 0)),
      out_shape=jax.ShapeDtypeStruct((n_tgt, d_out), F32),
  )(summed4, cnt32, h0c, w_l, w_r, b, att, w_out, b_out)


def kernel(x, edge_index_0, edge_index_1, W_l0, b0, W_r0, W_l1, b1, W_r1,
           att, W_out, b_out):
  x4 = x.reshape(4 * _N0, _G)
  s0, c0 = _agg0(x4, edge_index_0[0], edge_index_0[1])
  h0 = _sage_dense(s0, c0.reshape(32, _N1), x[:_N1], W_l0, W_r0,
                   b0.reshape(1, _D), bm=512)
  h04 = h0.reshape(4 * _N1, _G)
  s1, c1 = _agg1(h04, edge_index_1[0], edge_index_1[1])
  return _final(s1, c1.reshape(32, _N2), h0[:_N2], W_l1, W_r1,
                b1.reshape(1, _D), att.reshape(1, _D), W_out,
                b_out.reshape(1, -1), bm=512)
